# baseline (speedup 1.0000x reference)
import jax
import jax.numpy as jnp
from jax import lax
from jax.experimental import pallas as pl
from jax.experimental.pallas import tpu as pltpu
from jax.experimental.pallas import tpu_sc as plsc

B = 16384
C = 64
N = 1000000
PAIR = 2
NPAIR = N // PAIR
W = PAIR * C

NC = 2
NS = 16
NW = NC * NS
B_PER_W = B // NW
CHUNK = 32
N_CHUNKS = B_PER_W // CHUNK
NBUF = 2


def _sc_gather(conf_hbm, idx_hbm, out_hbm, idx_v, row_v, sems):
    wid = lax.axis_index("s") * NC + lax.axis_index("c")
    base = wid * B_PER_W
    pltpu.sync_copy(idx_hbm.at[pl.ds(wid * N_CHUNKS, N_CHUNKS)], idx_v)

    def fire(j, buf):
        return pltpu.async_copy(conf_hbm.at[idx_v.at[j]], row_v.at[buf], sems.at[buf])

    fire(0, 0)

    def body(j, _):
        buf = lax.rem(j, NBUF)
        nxt = lax.rem(j + 1, NBUF)

        @pl.when(j + 1 < N_CHUNKS)
        def _():
            fire(j + 1, nxt)

        pltpu.make_async_copy(conf_hbm.at[idx_v.at[j]], row_v.at[buf], sems.at[buf]).wait()
        pltpu.sync_copy(row_v.at[buf], out_hbm.at[pl.ds(base + j * CHUNK, CHUNK)])
        return ()

    lax.fori_loop(0, N_CHUNKS, body, (), unroll=False)


def _gather_pairs(conf2, idx2d):
    mesh = plsc.VectorSubcoreMesh(core_axis_name="c", subcore_axis_name="s")
    f = pl.kernel(
        _sc_gather,
        out_type=jax.ShapeDtypeStruct((B, W), jnp.float32),
        mesh=mesh,
        scratch_types=[
            pltpu.VMEM((N_CHUNKS, CHUNK), jnp.int32),
            pltpu.VMEM((NBUF, CHUNK, W), jnp.float32),
            pltpu.SemaphoreType.DMA((NBUF,)),
        ],
    )
    return f(conf2, idx2d)


TC_BLK = 1024


def _tc_loss(x_ref, pair_ref, par_ref, loss_ref, mean_ref):
    i = pl.program_id(0)
    x = x_ref[...]
    par = par_ref[...]
    conf = jnp.where(par == 0, pair_ref[:, :C], pair_ref[:, C:])
    m = jnp.max(x, axis=1, keepdims=True)
    lse = m + jnp.log(jnp.sum(jnp.exp(x - m), axis=1, keepdims=True))
    logsm = x - lse
    loss = -jnp.sum(logsm * conf, axis=1, keepdims=True)
    loss_ref[...] = loss

    @pl.when(i == 0)
    def _():
        mean_ref[...] = jnp.zeros_like(mean_ref)

    mean_ref[...] += jnp.sum(loss) * (1.0 / B)


def kernel(outputs, index, confidence):
    index = index.astype(jnp.int32)
    idx2d = (index // PAIR).reshape(B // CHUNK, CHUNK)
    par_in = (index % PAIR).reshape(B, 1)
    conf2 = confidence.reshape(NPAIR, W)
    pairs = _gather_pairs(conf2, idx2d)
    loss2d, mean2d = pl.pallas_call(
        _tc_loss,
        grid=(B // TC_BLK,),
        in_specs=[
            pl.BlockSpec((TC_BLK, C), lambda i: (i, 0)),
            pl.BlockSpec((TC_BLK, W), lambda i: (i, 0)),
            pl.BlockSpec((TC_BLK, 1), lambda i: (i, 0)),
        ],
        out_specs=[
            pl.BlockSpec((TC_BLK, 1), lambda i: (i, 0)),
            pl.BlockSpec((1, 1), lambda i: (0, 0)),
        ],
        out_shape=[
            jax.ShapeDtypeStruct((B, 1), jnp.float32),
            jax.ShapeDtypeStruct((1, 1), jnp.float32),
        ],
    )(outputs, pairs, par_in)
    return (mean2d[0, 0], loss2d.reshape(B))


# speedup vs baseline: 1.0026x; 1.0026x over previous
import jax
import jax.numpy as jnp
from jax import lax
from jax.experimental import pallas as pl
from jax.experimental.pallas import tpu as pltpu
from jax.experimental.pallas import tpu_sc as plsc

B = 16384
C = 64
N = 1000000
PAIR = 2
NPAIR = N // PAIR
W = PAIR * C

NC = 2
NS = 16
NW = NC * NS
B_PER_W = B // NW
CHUNK = 128
N_CHUNKS = B_PER_W // CHUNK


def _sc_gather(conf_hbm, idx_hbm, out_hbm, idx_v, row_v, sems):
    wid = lax.axis_index("s") * NC + lax.axis_index("c")
    base = wid * B_PER_W
    pltpu.sync_copy(idx_hbm.at[pl.ds(wid * N_CHUNKS, N_CHUNKS)], idx_v)

    for j in range(N_CHUNKS):
        pltpu.async_copy(
            conf_hbm.at[idx_v.at[j]],
            row_v.at[pl.ds(j * CHUNK, CHUNK)],
            sems.at[j],
        )
    for j in range(N_CHUNKS):
        pltpu.make_async_copy(
            conf_hbm.at[idx_v.at[j]],
            row_v.at[pl.ds(j * CHUNK, CHUNK)],
            sems.at[j],
        ).wait()
    pltpu.sync_copy(row_v, out_hbm.at[pl.ds(base, B_PER_W)])


def _gather_pairs(conf2, idx2d):
    mesh = plsc.VectorSubcoreMesh(core_axis_name="c", subcore_axis_name="s")
    f = pl.kernel(
        _sc_gather,
        out_type=jax.ShapeDtypeStruct((B, W), jnp.float32),
        mesh=mesh,
        scratch_types=[
            pltpu.VMEM((N_CHUNKS, CHUNK), jnp.int32),
            pltpu.VMEM((B_PER_W, W), jnp.float32),
            pltpu.SemaphoreType.DMA((N_CHUNKS,)),
        ],
    )
    return f(conf2, idx2d)


TC_BLK = 1024


def _tc_loss(x_ref, pair_ref, par_ref, loss_ref, mean_ref):
    i = pl.program_id(0)
    x = x_ref[...]
    par = par_ref[...]
    conf = jnp.where(par == 0, pair_ref[:, :C], pair_ref[:, C:])
    m = jnp.max(x, axis=1, keepdims=True)
    lse = m + jnp.log(jnp.sum(jnp.exp(x - m), axis=1, keepdims=True))
    logsm = x - lse
    loss = -jnp.sum(logsm * conf, axis=1, keepdims=True)
    loss_ref[...] = loss

    @pl.when(i == 0)
    def _():
        mean_ref[...] = jnp.zeros_like(mean_ref)

    mean_ref[...] += jnp.sum(loss) * (1.0 / B)


def kernel(outputs, index, confidence):
    index = index.astype(jnp.int32)
    idx2d = (index // PAIR).reshape(B // CHUNK, CHUNK)
    par_in = (index % PAIR).reshape(B, 1)
    conf2 = confidence.reshape(NPAIR, W)
    pairs = _gather_pairs(conf2, idx2d)
    loss2d, mean2d = pl.pallas_call(
        _tc_loss,
        grid=(B // TC_BLK,),
        in_specs=[
            pl.BlockSpec((TC_BLK, C), lambda i: (i, 0)),
            pl.BlockSpec((TC_BLK, W), lambda i: (i, 0)),
            pl.BlockSpec((TC_BLK, 1), lambda i: (i, 0)),
        ],
        out_specs=[
            pl.BlockSpec((TC_BLK, 1), lambda i: (i, 0)),
            pl.BlockSpec((1, 1), lambda i: (0, 0)),
        ],
        out_shape=[
            jax.ShapeDtypeStruct((B, 1), jnp.float32),
            jax.ShapeDtypeStruct((1, 1), jnp.float32),
        ],
    )(outputs, pairs, par_in)
    return (mean2d[0, 0], loss2d.reshape(B))


# speedup vs baseline: 1.6588x; 1.6545x over previous
import jax
import jax.numpy as jnp
from jax import lax
from jax.experimental import pallas as pl
from jax.experimental.pallas import tpu as pltpu
from jax.experimental.pallas import tpu_sc as plsc

B = 16384
C = 64
N = 1000000

NC = 2
NS = 16
NW = NC * NS
B_PER_W = B // NW
DEPTH = 32


def _sc_gather(conf_hbm, idx_hbm, out_hbm, idx_v, row_v, sem):
    wid = lax.axis_index("s") * NC + lax.axis_index("c")
    base = wid * B_PER_W
    pltpu.sync_copy(idx_hbm.at[pl.ds(base, B_PER_W)], idx_v)

    def fire(j):
        idx = idx_v[pl.ds(j, 1)][0]
        pltpu.async_copy(conf_hbm.at[pl.ds(idx, 1)], row_v.at[pl.ds(j, 1)], sem)

    def wait(j):
        pltpu.make_async_copy(
            conf_hbm.at[pl.ds(0, 1)], row_v.at[pl.ds(j, 1)], sem
        ).wait()

    def body(j, _):
        fire(j)

        @pl.when(j >= DEPTH)
        def _():
            wait(j - DEPTH)

        return ()

    lax.fori_loop(0, B_PER_W, body, (), unroll=False)

    def drain(j, _):
        wait(j)
        return ()

    lax.fori_loop(B_PER_W - DEPTH, B_PER_W, drain, (), unroll=False)
    pltpu.sync_copy(row_v, out_hbm.at[pl.ds(base, B_PER_W)])


def _gather_rows(confidence, index):
    mesh = plsc.VectorSubcoreMesh(core_axis_name="c", subcore_axis_name="s")
    f = pl.kernel(
        _sc_gather,
        out_type=jax.ShapeDtypeStruct((B, C), jnp.float32),
        mesh=mesh,
        scratch_types=[
            pltpu.VMEM((B_PER_W,), jnp.int32),
            pltpu.VMEM((B_PER_W, C), jnp.float32),
            pltpu.SemaphoreType.DMA,
        ],
    )
    return f(confidence, index)


TC_BLK = 1024


def _tc_loss(x_ref, conf_ref, loss_ref, mean_ref):
    i = pl.program_id(0)
    x = x_ref[...]
    conf = conf_ref[...]
    m = jnp.max(x, axis=1, keepdims=True)
    lse = m + jnp.log(jnp.sum(jnp.exp(x - m), axis=1, keepdims=True))
    logsm = x - lse
    loss = -jnp.sum(logsm * conf, axis=1, keepdims=True)
    loss_ref[...] = loss

    @pl.when(i == 0)
    def _():
        mean_ref[...] = jnp.zeros_like(mean_ref)

    mean_ref[...] += jnp.sum(loss) * (1.0 / B)


def kernel(outputs, index, confidence):
    index = index.astype(jnp.int32)
    rows = _gather_rows(confidence, index)
    loss2d, mean2d = pl.pallas_call(
        _tc_loss,
        grid=(B // TC_BLK,),
        in_specs=[
            pl.BlockSpec((TC_BLK, C), lambda i: (i, 0)),
            pl.BlockSpec((TC_BLK, C), lambda i: (i, 0)),
        ],
        out_specs=[
            pl.BlockSpec((TC_BLK, 1), lambda i: (i, 0)),
            pl.BlockSpec((1, 1), lambda i: (0, 0)),
        ],
        out_shape=[
            jax.ShapeDtypeStruct((B, 1), jnp.float32),
            jax.ShapeDtypeStruct((1, 1), jnp.float32),
        ],
    )(outputs, rows)
    return (mean2d[0, 0], loss2d.reshape(B))
